# Initial kernel scaffold; baseline (speedup 1.0000x reference)
import jax
import jax.numpy as jnp
from jax.experimental import pallas as pl


def kernel(node_features, adjacency_list_0, adjacency_list_1, adjacency_list_2, node_to_graph, W_init, b_init, W_msg, b_msg, W_aggr, b_aggr, W_ff1, b_ff1, W_ff2, b_ff2, W_score_s, b_score_s, W_val_s, b_val_s, W_out_s, b_out_s, W_score_m, b_score_m, W_val_m, b_val_m, W_out_m, b_out_m, W_max, b_max):
    raise NotImplementedError("write your pallas kernel here")


# speedup vs baseline: 4.3257x; 4.3257x over previous
import functools

import jax
import jax.numpy as jnp
from jax import lax
from jax.experimental import pallas as pl
from jax.experimental.pallas import tpu as pltpu
from jax.experimental.pallas import tpu_sc as plsc

N = 10000
NPAD = 10240
HID = 128
LAYERS = 10
ET = 3
NW = 32
NPW = NPAD // NW
HALF = NPW // 2
C = 128
BM = 1024
GRID = NPAD // BM
BRO = 512
GRID_RO = NPAD // BRO
FBIG = 3e38


def _mm(a, b, prec=None):
    return lax.dot_general(a, b, (((1,), (0,)), ((), ())),
                           precision=prec,
                           preferred_element_type=jnp.float32)


def _init_body(nf_ref, wi_ref, bi_ref, wm_ref, bm_ref, h_ref, mh_ref):
    h = _mm(nf_ref[...], wi_ref[...]) + bi_ref[...]
    h_ref[...] = h
    for t in range(ET):
        mh_ref[t] = jnp.maximum(_mm(h, wm_ref[t]) + bm_ref[t][None, :], 0.0)


def _upd_body(with_msg, h_ref, a3_ref, inv_ref, wa_ref, ba_ref, w1_ref,
              b1_ref, w2_ref, b2_ref, *rest):
    if with_msg:
        wm_ref, bm_ref, ho_ref, mh_ref = rest
    else:
        (ho_ref,) = rest
    h = h_ref[...]
    s = a3_ref[:, 0:HID]
    mx = a3_ref[:, HID:2 * HID]
    mn = a3_ref[:, 2 * HID:3 * HID]
    mean = s * inv_ref[...]
    z = (_mm(s, wa_ref[0:HID]) + _mm(mean, wa_ref[HID:2 * HID])
         + _mm(mx, wa_ref[2 * HID:3 * HID]) + _mm(mn, wa_ref[3 * HID:4 * HID])
         + ba_ref[...])
    h1 = h + jnp.maximum(z, 0.0)
    f = jnp.maximum(_mm(h1, w1_ref[...]) + b1_ref[...], 0.0)
    h2 = h1 + _mm(f, w2_ref[...]) + b2_ref[...]
    ho_ref[...] = h2
    if with_msg:
        for t in range(ET):
            mh_ref[t] = jnp.maximum(_mm(h2, wm_ref[t]) + bm_ref[t][None, :], 0.0)


def _expand_mat(rows):
    r = lax.broadcasted_iota(jnp.int32, (rows, 768), 0)
    c = lax.broadcasted_iota(jnp.int32, (rows, 768), 1)
    return jnp.where(c // 64 == r, 1.0, 0.0).astype(jnp.float32)


def _ro1_body(x_ref, wsp_ref, bsp_ref, wvs_ref, bvs_ref, wmp_ref, bmp_ref,
              wvm_ref, bvm_ref, svs_ref, vm_ref, scm_ref):
    x = x_ref[...]
    ss = _mm(x, wsp_ref[...]) + bsp_ref[...]
    sig = 1.0 / (1.0 + jnp.exp(-ss))
    vs = _mm(x, wvs_ref[...]) + bvs_ref[...]
    svs_ref[...] = _mm(sig, _expand_mat(128), lax.Precision.HIGHEST) * vs
    scm = _mm(x, wmp_ref[...]) + bmp_ref[...]
    scm_ref[...] = scm[:, 0:16]
    vm_ref[...] = _mm(x, wvm_ref[...]) + bvm_ref[...]


def _ro2_body(scm_ref, g3_ref, smx_ref, den_ref, vm_ref, wv_ref):
    g = g3_ref[0, 0, :]
    oh = jnp.where(g[:, None] == lax.broadcasted_iota(jnp.int32, (BRO, 256), 1),
                   1.0, 0.0).astype(jnp.float32)
    smxn = _mm(oh, smx_ref[...], lax.Precision.HIGHEST)
    denn = _mm(oh, den_ref[...], lax.Precision.HIGHEST)
    w = jnp.exp(scm_ref[...] - smxn) / (denn + 1e-9)
    wv_ref[...] = _mm(w, _expand_mat(16), lax.Precision.HIGHEST) * vm_ref[...]


def _fin_body(ps_ref, pm_ref, px_ref, wos_ref, bos_ref, wom_ref, bom_ref,
              wmx_ref, bmx_ref, o_ref):
    o_ref[...] = (_mm(ps_ref[...], wos_ref[...]) + bos_ref[...]
                  + _mm(pm_ref[...], wom_ref[...]) + bom_ref[...]
                  + _mm(px_ref[...], wmx_ref[...]) + bmx_ref[...])


def _row_spec(d):
    return pl.BlockSpec((BM, d), lambda i: (i, 0))


def _ro_spec(d):
    return pl.BlockSpec((BRO, d), lambda i: (i, 0))


def _full_spec(shape):
    nz = len(shape)
    return pl.BlockSpec(shape, lambda i: (0,) * nz)


def _wid():
    return lax.axis_index("s") * 2 + lax.axis_index("c")


def _sread(ref, i):
    return ref[pl.ds(i, 16)][0]


def _agg_body(mh_hbm, gidx_hbm, tgt_hbm, rptr_hbm, out_hbm,
              rptr_v, idx_v, tgt_v, rows_v, out_v, sem):
    wid = _wid()
    n0 = wid * NPW
    pltpu.sync_copy(rptr_hbm.at[pl.ds(n0, NPW + 8)],
                    rptr_v.at[pl.ds(0, NPW + 8)])
    zero = jnp.zeros((16,), jnp.float32)
    for half in range(2):
        hb = half * HALF

        def zrow(i, _):
            out_v[pl.ds(i * 16, 16)] = zero
            return 0
        lax.fori_loop(0, HALF * 384 // 16, zrow, 0)

        e0 = _sread(rptr_v, hb)
        e1 = _sread(rptr_v, hb + HALF)
        base = (e0 // 8) * 8
        nch = (e1 - base + C - 1) // C
        init = (jnp.int32(-1),) + tuple(zero for _ in range(24))

        def chunk(k, carry):
            coff = base + k * C
            pltpu.sync_copy(gidx_hbm.at[pl.ds(coff, C)], idx_v)
            pltpu.sync_copy(tgt_hbm.at[pl.ds(coff, C)], tgt_v.at[pl.ds(0, C)])
            pltpu.async_copy(mh_hbm.at[idx_v], rows_v, sem).wait()
            j0 = jnp.maximum(e0 - coff, 0)
            j1 = jnp.minimum(e1 - coff, C)

            def edge(j, ec):
                cur = ec[0]
                a = ec[1:]
                t = _sread(tgt_v, j)
                x = tuple(rows_v[j, pl.ds(kk * 16, 16)] for kk in range(8))
                changed = t != cur

                @pl.when(changed & (cur >= 0))
                def _():
                    off = (cur - (n0 + hb)) * 384
                    for kk in range(8):
                        out_v[pl.ds(off + kk * 16, 16)] = a[kk]
                        out_v[pl.ds(off + 128 + kk * 16, 16)] = a[8 + kk]
                        out_v[pl.ds(off + 256 + kk * 16, 16)] = a[16 + kk]

                return ((t,)
                        + tuple(jnp.where(changed, x[kk], a[kk] + x[kk])
                                for kk in range(8))
                        + tuple(jnp.where(changed, x[kk],
                                          jnp.maximum(a[8 + kk], x[kk]))
                                for kk in range(8))
                        + tuple(jnp.where(changed, x[kk],
                                          jnp.minimum(a[16 + kk], x[kk]))
                                for kk in range(8)))

            return lax.fori_loop(j0, j1, edge, carry)

        fin = lax.fori_loop(0, nch, chunk, init)
        cur = fin[0]

        @pl.when(cur >= 0)
        def _():
            off = (cur - (n0 + hb)) * 384
            for kk in range(8):
                out_v[pl.ds(off + kk * 16, 16)] = fin[1 + kk]
                out_v[pl.ds(off + 128 + kk * 16, 16)] = fin[9 + kk]
                out_v[pl.ds(off + 256 + kk * 16, 16)] = fin[17 + kk]

        pltpu.sync_copy(out_v,
                        out_hbm.at[pl.ds((n0 + hb) * 384, HALF * 384)])


def _stats_body(scm_hbm, grp_hbm, smx_hbm, den_hbm, grp_v, stage_v, osm_v,
                oden_v):
    wid = _wid()
    g0 = wid * 8
    pltpu.sync_copy(grp_hbm, grp_v.at[pl.ds(0, 264)])

    def graph(gi, _):
        a0 = _sread(grp_v, g0 + gi)
        a1 = _sread(grp_v, g0 + gi + 1)
        nn = a1 - a0
        nch = (nn + 63) // 64

        def p1(k, mx):
            pltpu.sync_copy(scm_hbm.at[pl.ds((a0 + k * 64) * 16, 1024)],
                            stage_v)
            j1 = jnp.minimum(nn - k * 64, 64)

            def b(j, m):
                return jnp.maximum(m, stage_v[pl.ds(j * 16, 16)])
            return lax.fori_loop(0, j1, b, mx)

        mx = lax.fori_loop(0, nch, p1, jnp.full((16,), -FBIG, jnp.float32))

        def p2(k, s):
            pltpu.sync_copy(scm_hbm.at[pl.ds((a0 + k * 64) * 16, 1024)],
                            stage_v)
            j1 = jnp.minimum(nn - k * 64, 64)

            def b(j, acc):
                return acc + jnp.exp(stage_v[pl.ds(j * 16, 16)] - mx)
            return lax.fori_loop(0, j1, b, s)

        den = lax.fori_loop(0, nch, p2, jnp.zeros((16,), jnp.float32))
        osm_v[pl.ds(gi * 16, 16)] = mx
        oden_v[pl.ds(gi * 16, 16)] = den
        return 0

    lax.fori_loop(0, 8, graph, 0)
    pltpu.sync_copy(osm_v, smx_hbm.at[pl.ds(g0 * 16, 128)])
    pltpu.sync_copy(oden_v, den_hbm.at[pl.ds(g0 * 16, 128)])


def _pool_body(svs_hbm, wvm_hbm, x_hbm, grp_hbm, ps_hbm, pm_hbm, px_hbm,
               grp_v, ss_v, sm_v, sx_v, acc_v):
    wid = _wid()
    g0 = wid * 8
    pltpu.sync_copy(grp_hbm, grp_v.at[pl.ds(0, 264)])
    zero = jnp.zeros((16,), jnp.float32)
    nbig = jnp.full((16,), -FBIG, jnp.float32)

    def graph(gi, _):
        a0 = _sread(grp_v, g0 + gi)
        a1 = _sread(grp_v, g0 + gi + 1)
        nn = a1 - a0
        nch = (nn + 31) // 32

        def zs(cc, _):
            acc_v[pl.ds(cc * 16, 16)] = zero
            return 0
        lax.fori_loop(0, 96, zs, 0)

        def zm(cc, _):
            acc_v[pl.ds(1536 + cc * 16, 16)] = nbig
            return 0
        lax.fori_loop(0, 88, zm, 0)

        def ch(k, _):
            r0 = a0 + k * 32
            pltpu.sync_copy(svs_hbm.at[pl.ds(r0 * 768, 32 * 768)], ss_v)
            pltpu.sync_copy(wvm_hbm.at[pl.ds(r0 * 768, 32 * 768)], sm_v)
            pltpu.sync_copy(x_hbm.at[pl.ds(r0 * 1408, 32 * 1408)], sx_v)
            j1 = jnp.minimum(nn - k * 32, 32)

            def fs(cc, _):
                def bs(j, a):
                    return a + ss_v[pl.ds(j * 768 + cc * 16, 16)]
                acc_v[pl.ds(cc * 16, 16)] = lax.fori_loop(
                    0, j1, bs, acc_v[pl.ds(cc * 16, 16)])
                return 0
            lax.fori_loop(0, 48, fs, 0)

            def fm(cc, _):
                def bm_(j, a):
                    return a + sm_v[pl.ds(j * 768 + cc * 16, 16)]
                acc_v[pl.ds(768 + cc * 16, 16)] = lax.fori_loop(
                    0, j1, bm_, acc_v[pl.ds(768 + cc * 16, 16)])
                return 0
            lax.fori_loop(0, 48, fm, 0)

            def fx(cc, _):
                def bx(j, a):
                    return jnp.maximum(a, sx_v[pl.ds(j * 1408 + cc * 16, 16)])
                acc_v[pl.ds(1536 + cc * 16, 16)] = lax.fori_loop(
                    0, j1, bx, acc_v[pl.ds(1536 + cc * 16, 16)])
                return 0
            lax.fori_loop(0, 88, fx, 0)
            return 0

        lax.fori_loop(0, nch, ch, 0)

        @pl.when(nn == 0)
        def _():
            def zx(cc, _):
                acc_v[pl.ds(1536 + cc * 16, 16)] = zero
                return 0
            lax.fori_loop(0, 88, zx, 0)

        pltpu.sync_copy(acc_v.at[pl.ds(0, 768)],
                        ps_hbm.at[pl.ds((g0 + gi) * 768, 768)])
        pltpu.sync_copy(acc_v.at[pl.ds(768, 768)],
                        pm_hbm.at[pl.ds((g0 + gi) * 768, 768)])
        pltpu.sync_copy(acc_v.at[pl.ds(1536, 1408)],
                        px_hbm.at[pl.ds((g0 + gi) * 1408, 1408)])
        return 0

    lax.fori_loop(0, 8, graph, 0)


@functools.lru_cache(maxsize=1)
def _sc_kernels():
    mesh = plsc.VectorSubcoreMesh(core_axis_name="c", subcore_axis_name="s")

    agg = pl.kernel(
        _agg_body, mesh=mesh,
        out_type=jax.ShapeDtypeStruct((NPAD * 384,), jnp.float32),
        scratch_types=[
            pltpu.VMEM((NPW + 24,), jnp.int32),
            pltpu.VMEM((C,), jnp.int32),
            pltpu.VMEM((C + 16,), jnp.int32),
            pltpu.VMEM((C, HID), jnp.float32),
            pltpu.VMEM((HALF * 384,), jnp.float32),
            pltpu.SemaphoreType.DMA,
        ],
    )

    stats = pl.kernel(
        _stats_body, mesh=mesh,
        out_type=[jax.ShapeDtypeStruct((256 * 16,), jnp.float32),
                  jax.ShapeDtypeStruct((256 * 16,), jnp.float32)],
        scratch_types=[
            pltpu.VMEM((280,), jnp.int32),
            pltpu.VMEM((1024,), jnp.float32),
            pltpu.VMEM((128,), jnp.float32),
            pltpu.VMEM((128,), jnp.float32),
        ],
    )

    pool = pl.kernel(
        _pool_body, mesh=mesh,
        out_type=[jax.ShapeDtypeStruct((256 * 768,), jnp.float32),
                  jax.ShapeDtypeStruct((256 * 768,), jnp.float32),
                  jax.ShapeDtypeStruct((256 * 1408,), jnp.float32)],
        scratch_types=[
            pltpu.VMEM((280,), jnp.int32),
            pltpu.VMEM((32 * 768,), jnp.float32),
            pltpu.VMEM((32 * 768,), jnp.float32),
            pltpu.VMEM((32 * 1408,), jnp.float32),
            pltpu.VMEM((2944,), jnp.float32),
        ],
    )
    return agg, stats, pool


def kernel(node_features, adjacency_list_0, adjacency_list_1,
           adjacency_list_2, node_to_graph,
           W_init, b_init, W_msg, b_msg, W_aggr, b_aggr, W_ff1, b_ff1,
           W_ff2, b_ff2, W_score_s, b_score_s, W_val_s, b_val_s, W_out_s,
           b_out_s, W_score_m, b_score_m, W_val_m, b_val_m, W_out_m,
           b_out_m, W_max, b_max):
    f32 = jnp.float32

    srcs, tgts = [], []
    for t, a in enumerate((adjacency_list_0, adjacency_list_1,
                           adjacency_list_2)):
        srcs.append(jnp.concatenate([a[:, 0], a[:, 1]]) + t * NPAD)
        tgts.append(jnp.concatenate([a[:, 1], a[:, 0]]))
    gsrc = jnp.concatenate(srcs)
    tgt_all = jnp.concatenate(tgts)
    E = gsrc.shape[0]
    order = jnp.argsort(tgt_all)
    gidx = gsrc[order].astype(jnp.int32)
    tgt_sorted = tgt_all[order].astype(jnp.int32)
    rptr = jnp.searchsorted(tgt_sorted, jnp.arange(NPAD + 1)).astype(jnp.int32)
    rptr = jnp.pad(rptr, (0, 7), constant_values=E)
    epad = ((E + C + 7) // C + 1) * C
    gidx = jnp.pad(gidx, (0, epad - E))
    tgt_sorted = jnp.pad(tgt_sorted, (0, epad - E))
    cnt = (rptr[1:NPAD + 1] - rptr[:NPAD]).astype(f32)
    inv = (1.0 / jnp.maximum(cnt, 1.0)).reshape(NPAD, 1)
    grp = jnp.searchsorted(node_to_graph, jnp.arange(257)).astype(jnp.int32)
    grp = jnp.pad(grp, (0, 7), constant_values=N)

    nf = jnp.pad(node_features, ((0, NPAD - N), (0, 0)))
    b2 = lambda b: b.reshape(1, -1)
    _agg, _stats, _pool = _sc_kernels()

    h, mh = pl.pallas_call(
        _init_body,
        grid=(GRID,),
        in_specs=[_row_spec(HID), _full_spec((HID, HID)), _full_spec((1, HID)),
                  _full_spec((ET, HID, HID)), _full_spec((ET, HID))],
        out_specs=[_row_spec(HID),
                   pl.BlockSpec((ET, BM, HID), lambda i: (0, i, 0))],
        out_shape=[jax.ShapeDtypeStruct((NPAD, HID), f32),
                   jax.ShapeDtypeStruct((ET, NPAD, HID), f32)],
    )(nf, W_init, b2(b_init), W_msg[0], b_msg[0])

    states = [h]
    upd_in_specs = [
        _row_spec(HID), _row_spec(384), _row_spec(1),
        _full_spec((4 * HID, HID)), _full_spec((1, HID)),
        _full_spec((HID, 1024)), _full_spec((1, 1024)),
        _full_spec((1024, HID)), _full_spec((1, HID)),
    ]
    msg_specs = [_full_spec((ET, HID, HID)), _full_spec((ET, HID))]

    for l in range(LAYERS):
        a3 = _agg(mh.reshape(ET * NPAD, HID), gidx, tgt_sorted,
                  rptr).reshape(NPAD, 384)
        last = l == LAYERS - 1
        if not last:
            h, mh = pl.pallas_call(
                functools.partial(_upd_body, True),
                grid=(GRID,),
                in_specs=upd_in_specs + msg_specs,
                out_specs=[_row_spec(HID),
                           pl.BlockSpec((ET, BM, HID), lambda i: (0, i, 0))],
                out_shape=[jax.ShapeDtypeStruct((NPAD, HID), f32),
                           jax.ShapeDtypeStruct((ET, NPAD, HID), f32)],
            )(h, a3, inv, W_aggr[l], b2(b_aggr[l]), W_ff1[l], b2(b_ff1[l]),
              W_ff2[l], b2(b_ff2[l]), W_msg[l + 1], b_msg[l + 1])
        else:
            h = pl.pallas_call(
                functools.partial(_upd_body, False),
                grid=(GRID,),
                in_specs=upd_in_specs,
                out_specs=_row_spec(HID),
                out_shape=jax.ShapeDtypeStruct((NPAD, HID), f32),
            )(h, a3, inv, W_aggr[l], b2(b_aggr[l]), W_ff1[l], b2(b_ff1[l]),
              W_ff2[l], b2(b_ff2[l]))
        states.append(h)

    x = jnp.concatenate(states, axis=-1)

    wsp = jnp.pad(W_score_s, ((0, 0), (0, 116)))
    bsp = jnp.pad(b_score_s, (0, 116))
    wmp = jnp.pad(W_score_m, ((0, 0), (0, 116)))
    bmp = jnp.concatenate([b_score_m, jnp.full((116,), -1e30, f32)])

    svs, vm, scm = pl.pallas_call(
        _ro1_body,
        grid=(GRID_RO,),
        in_specs=[_ro_spec(1408), _full_spec((1408, 128)), _full_spec((1, 128)),
                  _full_spec((1408, 768)), _full_spec((1, 768)),
                  _full_spec((1408, 128)), _full_spec((1, 128)),
                  _full_spec((1408, 768)), _full_spec((1, 768))],
        out_specs=[_ro_spec(768), _ro_spec(768), _ro_spec(16)],
        out_shape=[jax.ShapeDtypeStruct((NPAD, 768), f32),
                   jax.ShapeDtypeStruct((NPAD, 768), f32),
                   jax.ShapeDtypeStruct((NPAD, 16), f32)],
    )(x, wsp, b2(bsp), W_val_s, b2(b_val_s), wmp, b2(bmp), W_val_m,
      b2(b_val_m))

    smx, den = _stats(scm.reshape(-1), grp)
    smx = smx.reshape(256, 16)
    den = den.reshape(256, 16)

    g3 = jnp.pad(node_to_graph, (0, NPAD - N)).reshape(GRID_RO, 1, BRO)
    wv = pl.pallas_call(
        _ro2_body,
        grid=(GRID_RO,),
        in_specs=[_ro_spec(16), pl.BlockSpec((1, 1, BRO), lambda i: (i, 0, 0)),
                  _full_spec((256, 16)), _full_spec((256, 16)),
                  _ro_spec(768)],
        out_specs=_ro_spec(768),
        out_shape=jax.ShapeDtypeStruct((NPAD, 768), f32),
    )(scm, g3, smx, den, vm)

    ps, pm, px = _pool(svs.reshape(-1), wv.reshape(-1), x.reshape(-1), grp)
    ps = ps.reshape(256, 768)
    pm = pm.reshape(256, 768)
    px = px.reshape(256, 1408)

    out = pl.pallas_call(
        _fin_body,
        out_shape=jax.ShapeDtypeStruct((256, 512), f32),
    )(ps, pm, px, W_out_s, b2(b_out_s), W_out_m, b2(b_out_m), W_max,
      b2(b_max))
    return out


# speedup vs baseline: 7.7566x; 1.7932x over previous
import functools

import jax
import jax.numpy as jnp
from jax import lax
from jax.experimental import pallas as pl
from jax.experimental.pallas import tpu as pltpu
from jax.experimental.pallas import tpu_sc as plsc

N = 10000
NPAD = 10240
HID = 128
LAYERS = 10
ET = 3
NW = 32
NPW = NPAD // NW
HALF = NPW // 2
C = 128
BM = 1024
GRID = NPAD // BM
BRO = 512
GRID_RO = NPAD // BRO
FBIG = 3e38


def _mm(a, b, prec=None):
    return lax.dot_general(a, b, (((1,), (0,)), ((), ())),
                           precision=prec,
                           preferred_element_type=jnp.float32)


def _init_body(nf_ref, wi_ref, bi_ref, wm_ref, bm_ref, h_ref, mh_ref):
    h = _mm(nf_ref[...], wi_ref[...]) + bi_ref[...]
    h_ref[...] = h
    for t in range(ET):
        mh_ref[t] = jnp.maximum(_mm(h, wm_ref[t]) + bm_ref[t][None, :], 0.0)


def _upd_body(with_msg, h_ref, a3_ref, wa_ref, ba_ref, w1_ref,
              b1_ref, w2_ref, b2_ref, *rest):
    if with_msg:
        wm_ref, bm_ref, ho_ref, mh_ref = rest
    else:
        (ho_ref,) = rest
    h = h_ref[...]
    s = a3_ref[:, 0:HID]
    mx = a3_ref[:, HID:2 * HID]
    mn = a3_ref[:, 2 * HID:3 * HID]
    inv = 1.0 / jnp.maximum(a3_ref[:, 3 * HID:3 * HID + 1], 1.0)
    mean = s * inv
    z = (_mm(s, wa_ref[0:HID]) + _mm(mean, wa_ref[HID:2 * HID])
         + _mm(mx, wa_ref[2 * HID:3 * HID]) + _mm(mn, wa_ref[3 * HID:4 * HID])
         + ba_ref[...])
    h1 = h + jnp.maximum(z, 0.0)
    f = jnp.maximum(_mm(h1, w1_ref[...]) + b1_ref[...], 0.0)
    h2 = h1 + _mm(f, w2_ref[...]) + b2_ref[...]
    ho_ref[...] = h2
    if with_msg:
        for t in range(ET):
            mh_ref[t] = jnp.maximum(_mm(h2, wm_ref[t]) + bm_ref[t][None, :], 0.0)


def _expand_mat(rows):
    r = lax.broadcasted_iota(jnp.int32, (rows, 768), 0)
    c = lax.broadcasted_iota(jnp.int32, (rows, 768), 1)
    return jnp.where(c // 64 == r, 1.0, 0.0).astype(jnp.float32)


def _ro1_body(x_ref, wsp_ref, bsp_ref, wvs_ref, bvs_ref, wmp_ref, bmp_ref,
              wvm_ref, bvm_ref, svs_ref, vm_ref, scm_ref):
    x = x_ref[...]
    ss = _mm(x, wsp_ref[...]) + bsp_ref[...]
    sig = 1.0 / (1.0 + jnp.exp(-ss))
    vs = _mm(x, wvs_ref[...]) + bvs_ref[...]
    svs_ref[...] = _mm(sig, _expand_mat(128), lax.Precision.HIGHEST) * vs
    scm = _mm(x, wmp_ref[...]) + bmp_ref[...]
    scm_ref[...] = scm[:, 0:16]
    vm_ref[...] = _mm(x, wvm_ref[...]) + bvm_ref[...]


def _ro2_body(scm_ref, g3_ref, smx_ref, den_ref, vm_ref, wv_ref):
    g = g3_ref[0, 0, :]
    oh = jnp.where(g[:, None] == lax.broadcasted_iota(jnp.int32, (BRO, 256), 1),
                   1.0, 0.0).astype(jnp.float32)
    smxn = _mm(oh, smx_ref[...], lax.Precision.HIGHEST)
    denn = _mm(oh, den_ref[...], lax.Precision.HIGHEST)
    w = jnp.exp(scm_ref[...] - smxn) / (denn + 1e-9)
    wv_ref[...] = _mm(w, _expand_mat(16), lax.Precision.HIGHEST) * vm_ref[...]


def _fin_body(ps_ref, pm_ref, px_ref, wos_ref, bos_ref, wom_ref, bom_ref,
              wmx_ref, bmx_ref, o_ref):
    o_ref[...] = (_mm(ps_ref[...], wos_ref[...]) + bos_ref[...]
                  + _mm(pm_ref[...], wom_ref[...]) + bom_ref[...]
                  + _mm(px_ref[...], wmx_ref[...]) + bmx_ref[...])


def _row_spec(d):
    return pl.BlockSpec((BM, d), lambda i: (i, 0))


def _ro_spec(d):
    return pl.BlockSpec((BRO, d), lambda i: (i, 0))


def _full_spec(shape):
    nz = len(shape)
    return pl.BlockSpec(shape, lambda i: (0,) * nz)


def _wid():
    return lax.axis_index("s") * 2 + lax.axis_index("c")


def _sread(ref, i):
    return ref[pl.ds(i, 16)][0]


def _agg_body(mh_hbm, gidx_hbm, tgt_hbm, spans_hbm, out_hbm,
              spans_v, idx_v, tgt_v, rows_v, out_v, sem):
    wid = _wid()
    n0 = wid * NPW
    pltpu.sync_copy(spans_hbm, spans_v.at[pl.ds(0, 72)])
    zero = jnp.zeros((16,), jnp.float32)
    for half in range(2):
        hb = half * HALF

        def zrow(i, _):
            out_v[pl.ds(i * 16, 16)] = zero
            return 0
        lax.fori_loop(0, HALF * 400 // 16, zrow, 0)

        e0 = _sread(spans_v, 2 * wid + half)
        e1 = _sread(spans_v, 2 * wid + half + 1)
        base = (e0 // 8) * 8
        nch = (e1 - base + C - 1) // C
        init = (jnp.int32(-1), jnp.float32(0)) + tuple(zero for _ in range(24))

        def chunk(k, carry):
            coff = base + k * C
            pltpu.sync_copy(gidx_hbm.at[pl.ds(coff, C)], idx_v)
            pltpu.sync_copy(tgt_hbm.at[pl.ds(coff, C)], tgt_v.at[pl.ds(0, C)])
            pltpu.async_copy(mh_hbm.at[idx_v], rows_v, sem).wait()
            j0 = jnp.maximum(e0 - coff, 0)
            j1 = jnp.minimum(e1 - coff, C)

            def edge(j, ec):
                cur = ec[0]
                c = ec[1]
                a = ec[2:]
                t = _sread(tgt_v, j)
                x = tuple(rows_v[j, pl.ds(kk * 16, 16)] for kk in range(8))
                changed = t != cur

                @pl.when(changed & (cur >= 0))
                def _():
                    off = (cur - (n0 + hb)) * 400
                    for kk in range(8):
                        out_v[pl.ds(off + kk * 16, 16)] = a[kk]
                        out_v[pl.ds(off + 128 + kk * 16, 16)] = a[8 + kk]
                        out_v[pl.ds(off + 256 + kk * 16, 16)] = a[16 + kk]
                    out_v[pl.ds(off + 384, 16)] = jnp.broadcast_to(c, (16,))

                return ((t, jnp.where(changed, jnp.float32(1), c + 1))
                        + tuple(jnp.where(changed, x[kk], a[kk] + x[kk])
                                for kk in range(8))
                        + tuple(jnp.where(changed, x[kk],
                                          jnp.maximum(a[8 + kk], x[kk]))
                                for kk in range(8))
                        + tuple(jnp.where(changed, x[kk],
                                          jnp.minimum(a[16 + kk], x[kk]))
                                for kk in range(8)))

            return lax.fori_loop(j0, j1, edge, carry)

        fin = lax.fori_loop(0, nch, chunk, init)
        cur = fin[0]

        @pl.when(cur >= 0)
        def _():
            off = (cur - (n0 + hb)) * 400
            for kk in range(8):
                out_v[pl.ds(off + kk * 16, 16)] = fin[2 + kk]
                out_v[pl.ds(off + 128 + kk * 16, 16)] = fin[10 + kk]
                out_v[pl.ds(off + 256 + kk * 16, 16)] = fin[18 + kk]
            out_v[pl.ds(off + 384, 16)] = jnp.broadcast_to(fin[1], (16,))

        pltpu.sync_copy(out_v,
                        out_hbm.at[pl.ds((n0 + hb) * 400, HALF * 400)])


def _stats_body(scm_hbm, grp_hbm, smx_hbm, den_hbm, grp_v, stage_v, osm_v,
                oden_v):
    wid = _wid()
    g0 = wid * 8
    pltpu.sync_copy(grp_hbm, grp_v.at[pl.ds(0, 264)])

    def graph(gi, _):
        a0 = _sread(grp_v, g0 + gi)
        a1 = _sread(grp_v, g0 + gi + 1)
        nn = a1 - a0
        nch = (nn + 63) // 64

        def p1(k, mx):
            pltpu.sync_copy(scm_hbm.at[pl.ds((a0 + k * 64) * 16, 1024)],
                            stage_v)
            j1 = jnp.minimum(nn - k * 64, 64)

            def b(j, m):
                return jnp.maximum(m, stage_v[pl.ds(j * 16, 16)])
            return lax.fori_loop(0, j1, b, mx)

        mx = lax.fori_loop(0, nch, p1, jnp.full((16,), -FBIG, jnp.float32))

        def p2(k, s):
            pltpu.sync_copy(scm_hbm.at[pl.ds((a0 + k * 64) * 16, 1024)],
                            stage_v)
            j1 = jnp.minimum(nn - k * 64, 64)

            def b(j, acc):
                return acc + jnp.exp(stage_v[pl.ds(j * 16, 16)] - mx)
            return lax.fori_loop(0, j1, b, s)

        den = lax.fori_loop(0, nch, p2, jnp.zeros((16,), jnp.float32))
        osm_v[pl.ds(gi * 16, 16)] = mx
        oden_v[pl.ds(gi * 16, 16)] = den
        return 0

    lax.fori_loop(0, 8, graph, 0)
    pltpu.sync_copy(osm_v, smx_hbm.at[pl.ds(g0 * 16, 128)])
    pltpu.sync_copy(oden_v, den_hbm.at[pl.ds(g0 * 16, 128)])


def _pool_body(svs_hbm, wvm_hbm, x_hbm, grp_hbm, ps_hbm, pm_hbm, px_hbm,
               grp_v, ss_v, sm_v, sx_v, acc_v):
    wid = _wid()
    g0 = wid * 8
    pltpu.sync_copy(grp_hbm, grp_v.at[pl.ds(0, 264)])
    zero = jnp.zeros((16,), jnp.float32)
    nbig = jnp.full((16,), -FBIG, jnp.float32)

    def graph(gi, _):
        a0 = _sread(grp_v, g0 + gi)
        a1 = _sread(grp_v, g0 + gi + 1)
        nn = a1 - a0
        nch = (nn + 31) // 32

        def zs(cc, _):
            acc_v[pl.ds(cc * 16, 16)] = zero
            return 0
        lax.fori_loop(0, 96, zs, 0)

        def zm(cc, _):
            acc_v[pl.ds(1536 + cc * 16, 16)] = nbig
            return 0
        lax.fori_loop(0, 88, zm, 0)

        def ch(k, _):
            r0 = a0 + k * 32
            pltpu.sync_copy(svs_hbm.at[pl.ds(r0 * 768, 32 * 768)], ss_v)
            pltpu.sync_copy(wvm_hbm.at[pl.ds(r0 * 768, 32 * 768)], sm_v)
            pltpu.sync_copy(x_hbm.at[pl.ds(r0 * 1408, 32 * 1408)], sx_v)
            j1 = jnp.minimum(nn - k * 32, 32)

            def fs(cc, _):
                def bs(j, a):
                    return a + ss_v[pl.ds(j * 768 + cc * 16, 16)]
                acc_v[pl.ds(cc * 16, 16)] = lax.fori_loop(
                    0, j1, bs, acc_v[pl.ds(cc * 16, 16)])
                return 0
            lax.fori_loop(0, 48, fs, 0)

            def fm(cc, _):
                def bm_(j, a):
                    return a + sm_v[pl.ds(j * 768 + cc * 16, 16)]
                acc_v[pl.ds(768 + cc * 16, 16)] = lax.fori_loop(
                    0, j1, bm_, acc_v[pl.ds(768 + cc * 16, 16)])
                return 0
            lax.fori_loop(0, 48, fm, 0)

            def fx(cc, _):
                def bx(j, a):
                    return jnp.maximum(a, sx_v[pl.ds(j * 1408 + cc * 16, 16)])
                acc_v[pl.ds(1536 + cc * 16, 16)] = lax.fori_loop(
                    0, j1, bx, acc_v[pl.ds(1536 + cc * 16, 16)])
                return 0
            lax.fori_loop(0, 88, fx, 0)
            return 0

        lax.fori_loop(0, nch, ch, 0)

        @pl.when(nn == 0)
        def _():
            def zx(cc, _):
                acc_v[pl.ds(1536 + cc * 16, 16)] = zero
                return 0
            lax.fori_loop(0, 88, zx, 0)

        pltpu.sync_copy(acc_v.at[pl.ds(0, 768)],
                        ps_hbm.at[pl.ds((g0 + gi) * 768, 768)])
        pltpu.sync_copy(acc_v.at[pl.ds(768, 768)],
                        pm_hbm.at[pl.ds((g0 + gi) * 768, 768)])
        pltpu.sync_copy(acc_v.at[pl.ds(1536, 1408)],
                        px_hbm.at[pl.ds((g0 + gi) * 1408, 1408)])
        return 0

    lax.fori_loop(0, 8, graph, 0)


@functools.lru_cache(maxsize=1)
def _sc_kernels():
    mesh = plsc.VectorSubcoreMesh(core_axis_name="c", subcore_axis_name="s")

    agg = pl.kernel(
        _agg_body, mesh=mesh,
        out_type=jax.ShapeDtypeStruct((NPAD * 400,), jnp.float32),
        scratch_types=[
            pltpu.VMEM((88,), jnp.int32),
            pltpu.VMEM((C,), jnp.int32),
            pltpu.VMEM((C + 16,), jnp.int32),
            pltpu.VMEM((C, HID), jnp.float32),
            pltpu.VMEM((HALF * 400,), jnp.float32),
            pltpu.SemaphoreType.DMA,
        ],
    )

    stats = pl.kernel(
        _stats_body, mesh=mesh,
        out_type=[jax.ShapeDtypeStruct((256 * 16,), jnp.float32),
                  jax.ShapeDtypeStruct((256 * 16,), jnp.float32)],
        scratch_types=[
            pltpu.VMEM((280,), jnp.int32),
            pltpu.VMEM((1024,), jnp.float32),
            pltpu.VMEM((128,), jnp.float32),
            pltpu.VMEM((128,), jnp.float32),
        ],
    )

    pool = pl.kernel(
        _pool_body, mesh=mesh,
        out_type=[jax.ShapeDtypeStruct((256 * 768,), jnp.float32),
                  jax.ShapeDtypeStruct((256 * 768,), jnp.float32),
                  jax.ShapeDtypeStruct((256 * 1408,), jnp.float32)],
        scratch_types=[
            pltpu.VMEM((280,), jnp.int32),
            pltpu.VMEM((32 * 768,), jnp.float32),
            pltpu.VMEM((32 * 768,), jnp.float32),
            pltpu.VMEM((32 * 1408,), jnp.float32),
            pltpu.VMEM((2944,), jnp.float32),
        ],
    )
    return agg, stats, pool


def kernel(node_features, adjacency_list_0, adjacency_list_1,
           adjacency_list_2, node_to_graph,
           W_init, b_init, W_msg, b_msg, W_aggr, b_aggr, W_ff1, b_ff1,
           W_ff2, b_ff2, W_score_s, b_score_s, W_val_s, b_val_s, W_out_s,
           b_out_s, W_score_m, b_score_m, W_val_m, b_val_m, W_out_m,
           b_out_m, W_max, b_max):
    f32 = jnp.float32

    srcs, tgts = [], []
    for t, a in enumerate((adjacency_list_0, adjacency_list_1,
                           adjacency_list_2)):
        srcs.append(jnp.concatenate([a[:, 0], a[:, 1]]) + t * NPAD)
        tgts.append(jnp.concatenate([a[:, 1], a[:, 0]]))
    gsrc = jnp.concatenate(srcs).astype(jnp.int32)
    tgt_all = jnp.concatenate(tgts).astype(jnp.int32)
    E = gsrc.shape[0]
    tgt_sorted, gidx = lax.sort((tgt_all, gsrc), num_keys=1)
    bnd = (jnp.arange(65, dtype=jnp.int32) * HALF)
    spans = jnp.sum(tgt_all[None, :] < bnd[:, None], axis=1,
                    dtype=jnp.int32)
    spans = jnp.pad(spans, (0, 7), constant_values=E)
    grp = jnp.sum(node_to_graph[None, :]
                  < jnp.arange(257, dtype=jnp.int32)[:, None], axis=1,
                  dtype=jnp.int32)
    grp = jnp.pad(grp, (0, 7), constant_values=N)
    epad = ((E + C + 7) // C + 1) * C
    gidx = jnp.pad(gidx, (0, epad - E))
    tgt_sorted = jnp.pad(tgt_sorted, (0, epad - E))

    nf = jnp.pad(node_features, ((0, NPAD - N), (0, 0)))
    b2 = lambda b: b.reshape(1, -1)
    _agg, _stats, _pool = _sc_kernels()

    h, mh = pl.pallas_call(
        _init_body,
        grid=(GRID,),
        in_specs=[_row_spec(HID), _full_spec((HID, HID)), _full_spec((1, HID)),
                  _full_spec((ET, HID, HID)), _full_spec((ET, HID))],
        out_specs=[_row_spec(HID),
                   pl.BlockSpec((ET, BM, HID), lambda i: (0, i, 0))],
        out_shape=[jax.ShapeDtypeStruct((NPAD, HID), f32),
                   jax.ShapeDtypeStruct((ET, NPAD, HID), f32)],
    )(nf, W_init, b2(b_init), W_msg[0], b_msg[0])

    states = [h]
    upd_in_specs = [
        _row_spec(HID), _row_spec(400),
        _full_spec((4 * HID, HID)), _full_spec((1, HID)),
        _full_spec((HID, 1024)), _full_spec((1, 1024)),
        _full_spec((1024, HID)), _full_spec((1, HID)),
    ]
    msg_specs = [_full_spec((ET, HID, HID)), _full_spec((ET, HID))]

    for l in range(LAYERS):
        a3 = _agg(mh.reshape(ET * NPAD, HID), gidx, tgt_sorted,
                  spans).reshape(NPAD, 400)
        last = l == LAYERS - 1
        if not last:
            h, mh = pl.pallas_call(
                functools.partial(_upd_body, True),
                grid=(GRID,),
                in_specs=upd_in_specs + msg_specs,
                out_specs=[_row_spec(HID),
                           pl.BlockSpec((ET, BM, HID), lambda i: (0, i, 0))],
                out_shape=[jax.ShapeDtypeStruct((NPAD, HID), f32),
                           jax.ShapeDtypeStruct((ET, NPAD, HID), f32)],
            )(h, a3, W_aggr[l], b2(b_aggr[l]), W_ff1[l], b2(b_ff1[l]),
              W_ff2[l], b2(b_ff2[l]), W_msg[l + 1], b_msg[l + 1])
        else:
            h = pl.pallas_call(
                functools.partial(_upd_body, False),
                grid=(GRID,),
                in_specs=upd_in_specs,
                out_specs=_row_spec(HID),
                out_shape=jax.ShapeDtypeStruct((NPAD, HID), f32),
            )(h, a3, W_aggr[l], b2(b_aggr[l]), W_ff1[l], b2(b_ff1[l]),
              W_ff2[l], b2(b_ff2[l]))
        states.append(h)

    x = jnp.concatenate(states, axis=-1)

    wsp = jnp.pad(W_score_s, ((0, 0), (0, 116)))
    bsp = jnp.pad(b_score_s, (0, 116))
    wmp = jnp.pad(W_score_m, ((0, 0), (0, 116)))
    bmp = jnp.concatenate([b_score_m, jnp.full((116,), -1e30, f32)])

    svs, vm, scm = pl.pallas_call(
        _ro1_body,
        grid=(GRID_RO,),
        in_specs=[_ro_spec(1408), _full_spec((1408, 128)), _full_spec((1, 128)),
                  _full_spec((1408, 768)), _full_spec((1, 768)),
                  _full_spec((1408, 128)), _full_spec((1, 128)),
                  _full_spec((1408, 768)), _full_spec((1, 768))],
        out_specs=[_ro_spec(768), _ro_spec(768), _ro_spec(16)],
        out_shape=[jax.ShapeDtypeStruct((NPAD, 768), f32),
                   jax.ShapeDtypeStruct((NPAD, 768), f32),
                   jax.ShapeDtypeStruct((NPAD, 16), f32)],
    )(x, wsp, b2(bsp), W_val_s, b2(b_val_s), wmp, b2(bmp), W_val_m,
      b2(b_val_m))

    smx, den = _stats(scm.reshape(-1), grp)
    smx = smx.reshape(256, 16)
    den = den.reshape(256, 16)

    g3 = jnp.pad(node_to_graph, (0, NPAD - N)).reshape(GRID_RO, 1, BRO)
    wv = pl.pallas_call(
        _ro2_body,
        grid=(GRID_RO,),
        in_specs=[_ro_spec(16), pl.BlockSpec((1, 1, BRO), lambda i: (i, 0, 0)),
                  _full_spec((256, 16)), _full_spec((256, 16)),
                  _ro_spec(768)],
        out_specs=_ro_spec(768),
        out_shape=jax.ShapeDtypeStruct((NPAD, 768), f32),
    )(scm, g3, smx, den, vm)

    ps, pm, px = _pool(svs.reshape(-1), wv.reshape(-1), x.reshape(-1), grp)
    ps = ps.reshape(256, 768)
    pm = pm.reshape(256, 768)
    px = px.reshape(256, 1408)

    out = pl.pallas_call(
        _fin_body,
        out_shape=jax.ShapeDtypeStruct((256, 512), f32),
    )(ps, pm, px, W_out_s, b2(b_out_s), W_out_m, b2(b_out_m), W_max,
      b2(b_max))
    return out
